# Initial kernel scaffold; baseline (speedup 1.0000x reference)
import jax
import jax.numpy as jnp
from jax.experimental import pallas as pl


def kernel(pos1, pos2, feature1, feature2, W0, g0, b0, W1, g1, b1, W2, g2, b2, W3, g3, b3):
    raise NotImplementedError("write your pallas kernel here")


# speedup vs baseline: 10.1941x; 10.1941x over previous
import functools

import jax
import jax.numpy as jnp
from jax import lax
from jax.experimental import pallas as pl
from jax.experimental.pallas import tpu as pltpu
from jax.experimental.pallas import tpu_sc as plsc

B, N, M, S, C = 8, 2048, 2048, 16, 64
T = B * N * S
T4 = B * N
EPS = 1e-5
F = 128
F4 = 256

NT = 256


def _knn_body(p1t_ref, p2_ref, idx_ref):
    b = pl.program_id(0)
    p1t = p1t_ref[0]
    p2 = p2_ref[0]
    prod = jax.lax.dot_general(
        p1t.astype(jnp.bfloat16), p2.astype(jnp.bfloat16),
        (((1,), (0,)), ((), ())),
        preferred_element_type=jnp.float32)
    d = (jnp.sum(p1t * p1t, axis=1, keepdims=True) - 2.0 * prod
         + jnp.sum(p2 * p2, axis=0, keepdims=True))
    iota = lax.broadcasted_iota(jnp.int32, (NT, M), 1)
    cols = []
    for _ in range(S):
        dmin = jnp.min(d, axis=1, keepdims=True)
        imin = jnp.min(jnp.where(d == dmin, iota, M), axis=1, keepdims=True)
        cols.append(imin)
        d = jnp.where(iota == imin, jnp.inf, d)
    idx_ref[0] = jnp.concatenate(cols, axis=1) + b * M


def _knn(pos1_t, pos2):
    return pl.pallas_call(
        _knn_body,
        grid=(B, N // NT),
        in_specs=[
            pl.BlockSpec((1, NT, 3), lambda b, i: (b, i, 0)),
            pl.BlockSpec((1, 3, M), lambda b, i: (b, 0, 0)),
        ],
        out_specs=pl.BlockSpec((1, NT, S), lambda b, i: (b, i, 0)),
        out_shape=jax.ShapeDtypeStruct((B, N, S), jnp.int32),
    )(pos1_t, pos2)


def _g2h1_body(f2t_ref, p2t_ref, f1t_ref, p1t_ref, wf2_ref, wf1_ref, wp_ref,
               g2_ref, h1_ref):
    wp = wp_ref[...]
    g2_ref[0] = (jnp.dot(f2t_ref[0], wf2_ref[...],
                         preferred_element_type=jnp.float32)
                 + jnp.dot(p2t_ref[0], wp, preferred_element_type=jnp.float32))
    h1_ref[0] = (jnp.dot(f1t_ref[0], wf1_ref[...],
                         preferred_element_type=jnp.float32)
                 - jnp.dot(p1t_ref[0], wp, preferred_element_type=jnp.float32))


def _g2h1(f2t, p2t, f1t, p1t, wf2, wf1, wp):
    return pl.pallas_call(
        _g2h1_body,
        grid=(B,),
        in_specs=[
            pl.BlockSpec((1, M, C), lambda b: (b, 0, 0)),
            pl.BlockSpec((1, M, 3), lambda b: (b, 0, 0)),
            pl.BlockSpec((1, N, C), lambda b: (b, 0, 0)),
            pl.BlockSpec((1, N, 3), lambda b: (b, 0, 0)),
            pl.BlockSpec((C, F), lambda b: (0, 0)),
            pl.BlockSpec((C, F), lambda b: (0, 0)),
            pl.BlockSpec((3, F), lambda b: (0, 0)),
        ],
        out_specs=[
            pl.BlockSpec((1, M, F), lambda b: (b, 0, 0)),
            pl.BlockSpec((1, N, F), lambda b: (b, 0, 0)),
        ],
        out_shape=[
            jax.ShapeDtypeStruct((B, M, F), jnp.float32),
            jax.ShapeDtypeStruct((B, N, F), jnp.float32),
        ],
    )(f2t, p2t, f1t, p1t, wf2, wf1, wp)


NC, NS = 2, 16
NW = NC * NS
TPW = T // NW
KCH = 128
NCH = TPW // KCH


def _sc_gather_body(table_hbm, idx_hbm, out_hbm, idx_v, rows_v, sem):
    wid = lax.axis_index("s") * NC + lax.axis_index("c")
    base = wid * TPW

    def chunk(j, carry):
        off = base + j * KCH
        pltpu.sync_copy(idx_hbm.at[pl.ds(off, KCH)], idx_v)
        pltpu.async_copy(table_hbm.at[idx_v], rows_v, sem).wait()
        pltpu.sync_copy(rows_v, out_hbm.at[pl.ds(off, KCH)])
        return carry

    lax.fori_loop(0, NCH, chunk, 0)


def _sc_gather(table, gidx):
    kern = functools.partial(
        pl.kernel,
        out_type=jax.ShapeDtypeStruct((T, F), jnp.float32),
        mesh=plsc.VectorSubcoreMesh(core_axis_name="c", subcore_axis_name="s"),
        scratch_types=[
            pltpu.VMEM((KCH,), jnp.int32),
            pltpu.VMEM((KCH, F), jnp.float32),
            pltpu.SemaphoreType.DMA,
        ],
    )(_sc_gather_body)
    return kern(table, gidx)


RB = 2048


def _acc_stats(st_ref, z):
    s1 = jnp.sum(z, axis=0, keepdims=True)
    s2 = jnp.sum(z * z, axis=0, keepdims=True)
    acc = jnp.concatenate([s1, s2], axis=0)

    @pl.when(pl.program_id(0) == 0)
    def _():
        st_ref[...] = jnp.zeros_like(st_ref)

    st_ref[...] += acc


def _expand_h(h):
    return jnp.broadcast_to(h[:, None, :], (RB // S, S, F)).reshape(RB, F)


def _stats1_body(g_ref, h_ref, st_ref):
    z = g_ref[...] + _expand_h(h_ref[...])
    _acc_stats(st_ref, z)


def _stats1(rows, h1f):
    return pl.pallas_call(
        _stats1_body,
        grid=(T // RB,),
        in_specs=[
            pl.BlockSpec((RB, F), lambda i: (i, 0)),
            pl.BlockSpec((RB // S, F), lambda i: (i, 0)),
        ],
        out_specs=pl.BlockSpec((2, F), lambda i: (0, 0)),
        out_shape=jax.ShapeDtypeStruct((2, F), jnp.float32),
    )(rows, h1f)


def _layer2_body(g_ref, h_ref, ss_ref, w_ref, z_ref, st_ref):
    z1 = g_ref[...] + _expand_h(h_ref[...])
    x = jnp.maximum(z1 * ss_ref[0:1, :] + ss_ref[1:2, :], 0.0)
    z2 = jnp.dot(x, w_ref[...], preferred_element_type=jnp.float32)
    z_ref[...] = z2
    _acc_stats(st_ref, z2)


def _layer2(rows, h1f, ss, wt):
    return pl.pallas_call(
        _layer2_body,
        grid=(T // RB,),
        in_specs=[
            pl.BlockSpec((RB, F), lambda i: (i, 0)),
            pl.BlockSpec((RB // S, F), lambda i: (i, 0)),
            pl.BlockSpec((2, F), lambda i: (0, 0)),
            pl.BlockSpec((F, F), lambda i: (0, 0)),
        ],
        out_specs=[
            pl.BlockSpec((RB, F), lambda i: (i, 0)),
            pl.BlockSpec((2, F), lambda i: (0, 0)),
        ],
        out_shape=[
            jax.ShapeDtypeStruct((T, F), jnp.float32),
            jax.ShapeDtypeStruct((2, F), jnp.float32),
        ],
    )(rows, h1f, ss, wt)


def _layer3_body(z_ref, ss_ref, w_ref, zo_ref, st_ref):
    x = jnp.maximum(z_ref[...] * ss_ref[0:1, :] + ss_ref[1:2, :], 0.0)
    z = jnp.dot(x, w_ref[...], preferred_element_type=jnp.float32)
    zo_ref[...] = z
    _acc_stats(st_ref, z)


def _layer3(zin, ss, wt):
    return pl.pallas_call(
        _layer3_body,
        grid=(T // RB,),
        in_specs=[
            pl.BlockSpec((RB, F), lambda i: (i, 0)),
            pl.BlockSpec((2, F), lambda i: (0, 0)),
            pl.BlockSpec((F, F), lambda i: (0, 0)),
        ],
        out_specs=[
            pl.BlockSpec((RB, F), lambda i: (i, 0)),
            pl.BlockSpec((2, F), lambda i: (0, 0)),
        ],
        out_shape=[
            jax.ShapeDtypeStruct((T, F), jnp.float32),
            jax.ShapeDtypeStruct((2, F), jnp.float32),
        ],
    )(zin, ss, wt)


def _layer4_body(z_ref, ss_ref, w_ref, zo_ref, st_ref):
    x = jnp.maximum(z_ref[...] * ss_ref[0:1, :] + ss_ref[1:2, :], 0.0)
    xm = jnp.max(x.reshape(RB // S, S, F), axis=1)
    z4 = jnp.dot(xm, w_ref[...], preferred_element_type=jnp.float32)
    zo_ref[...] = z4
    _acc_stats(st_ref, z4)


def _layer4(zin, ss, wt):
    return pl.pallas_call(
        _layer4_body,
        grid=(T // RB,),
        in_specs=[
            pl.BlockSpec((RB, F), lambda i: (i, 0)),
            pl.BlockSpec((2, F), lambda i: (0, 0)),
            pl.BlockSpec((F, F4), lambda i: (0, 0)),
        ],
        out_specs=[
            pl.BlockSpec((RB // S, F4), lambda i: (i, 0)),
            pl.BlockSpec((2, F4), lambda i: (0, 0)),
        ],
        out_shape=[
            jax.ShapeDtypeStruct((T4, F4), jnp.float32),
            jax.ShapeDtypeStruct((2, F4), jnp.float32),
        ],
    )(zin, ss, wt)


RN = 512


def _final_body(z_ref, ss_ref, out_ref):
    y = jnp.maximum(z_ref[...] * ss_ref[0:1, :] + ss_ref[1:2, :], 0.0)
    out_ref[0] = y.T


def _final(z4, ss):
    return pl.pallas_call(
        _final_body,
        grid=(B, N // RN),
        in_specs=[
            pl.BlockSpec((RN, F4), lambda b, j: (b * (N // RN) + j, 0)),
            pl.BlockSpec((2, F4), lambda b, j: (0, 0)),
        ],
        out_specs=pl.BlockSpec((1, F4, RN), lambda b, j: (b, 0, j)),
        out_shape=jax.ShapeDtypeStruct((B, F4, N), jnp.float32),
    )(z4, ss)


def _scale_shift(stats, g, b, count):
    mean = stats[0] / count
    var = stats[1] / count - mean * mean
    scale = g / jnp.sqrt(var + EPS)
    return jnp.stack([scale, b - mean * scale])


def kernel(pos1, pos2, feature1, feature2,
           W0, g0, b0, W1, g1, b1, W2, g2, b2, W3, g3, b3):
    pos1_t = jnp.transpose(pos1, (0, 2, 1))
    pos2_t = jnp.transpose(pos2, (0, 2, 1))
    f1t = jnp.transpose(feature1, (0, 2, 1))
    f2t = jnp.transpose(feature2, (0, 2, 1))
    wp = jnp.transpose(W0[:, :3])
    wf2 = jnp.transpose(W0[:, 3:3 + C])
    wf1 = jnp.transpose(W0[:, 3 + C:])

    gidx = _knn(pos1_t, pos2)
    g2t, h1 = _g2h1(f2t, pos2_t, f1t, pos1_t, wf2, wf1, wp)

    rows = _sc_gather(g2t.reshape(B * M, F), gidx.reshape(T))
    h1f = h1.reshape(T4, F)

    st1 = _stats1(rows, h1f)
    ss1 = _scale_shift(st1, g0, b0, jnp.float32(T))
    z2, st2 = _layer2(rows, h1f, ss1, jnp.transpose(W1))
    ss2 = _scale_shift(st2, g1, b1, jnp.float32(T))
    z3, st3 = _layer3(z2, ss2, jnp.transpose(W2))
    ss3 = _scale_shift(st3, g2, b2, jnp.float32(T))
    z4, st4 = _layer4(z3, ss3, jnp.transpose(W3))
    ss4 = _scale_shift(st4, g3, b3, jnp.float32(T4))
    return _final(z4, ss4)
